# baseline (speedup 1.0000x reference)
import functools

import jax
import jax.numpy as jnp
from jax import lax
from jax.experimental import pallas as pl
from jax.experimental.pallas import tpu as pltpu
from jax.experimental.pallas import tpu_sc as plsc

VOCAB = 1000000
DIM = 64
SEQ_LEN = 200
BATCH = 4096

NUM_CORES = 2
NUM_SUBCORES = 16
NW = NUM_CORES * NUM_SUBCORES
ROWS = BATCH * SEQ_LEN
ROWS_PER_W = ROWS // NW
CHUNK = 4 * SEQ_LEN
CHUNKS_PER_W = ROWS_PER_W // CHUNK
GATHER = 100
GATHERS_PER_CHUNK = CHUNK // GATHER
IDX_COLS = GATHER
LANES = 16


def _pos_encoding():
    even_i = jnp.arange(0, DIM, 2).astype(jnp.float32)
    denominator = jnp.power(10000.0, even_i / DIM)
    position = jnp.arange(SEQ_LEN).reshape(SEQ_LEN, 1).astype(jnp.float32)
    even_pe = jnp.sin(position / denominator)
    odd_pe = jnp.cos(position / denominator)
    return jnp.stack([even_pe, odd_pe], axis=2).reshape(SEQ_LEN, DIM)


def _body(idx_hbm, pe_hbm, tab_hbm, out_hbm, idx_v, rows_v, pe_v, sem):
    wid = lax.axis_index("s") * NUM_CORES + lax.axis_index("c")
    pltpu.sync_copy(pe_hbm, pe_v)

    def chunk_body(c, carry):
        r0 = wid * (ROWS_PER_W // IDX_COLS) + c * GATHERS_PER_CHUNK
        pltpu.sync_copy(idx_hbm.at[pl.ds(r0, GATHERS_PER_CHUNK)], idx_v)
        cps = [
            pltpu.async_copy(
                tab_hbm.at[idx_v.at[j]],
                rows_v.at[pl.ds(j * GATHER, GATHER)],
                sem,
            )
            for j in range(GATHERS_PER_CHUNK)
        ]
        for cp in cps:
            cp.wait()

        def add_body(s, carry2):
            for rep in range(CHUNK // SEQ_LEN):
                row = rep * SEQ_LEN + s
                for d in range(DIM // LANES):
                    sl = pl.ds(d * LANES, LANES)
                    rows_v[row, sl] = rows_v[row, sl] + pe_v[s, sl]
            return carry2

        lax.fori_loop(0, SEQ_LEN, add_body, 0)
        pltpu.sync_copy(
            rows_v, out_hbm.at[pl.ds(wid * ROWS_PER_W + c * CHUNK, CHUNK)]
        )
        return carry

    lax.fori_loop(0, CHUNKS_PER_W, chunk_body, 0)


_emb_call = functools.partial(
    pl.kernel,
    mesh=plsc.VectorSubcoreMesh(core_axis_name="c", subcore_axis_name="s"),
    out_type=jax.ShapeDtypeStruct((ROWS, DIM), jnp.float32),
    scratch_types=[
        pltpu.VMEM((GATHERS_PER_CHUNK, IDX_COLS), jnp.int32),
        pltpu.VMEM((CHUNK, DIM), jnp.float32),
        pltpu.VMEM((SEQ_LEN, DIM), jnp.float32),
        pltpu.SemaphoreType.DMA,
    ],
    compiler_params=pltpu.CompilerParams(use_tc_tiling_on_sc=False),
)(_body)


@jax.jit
def kernel(inputs, word_embedding):
    idx = inputs.astype(jnp.int32).reshape(ROWS // IDX_COLS, IDX_COLS)
    pe = _pos_encoding()
    out = _emb_call(idx, pe, word_embedding)
    return out.reshape(BATCH, SEQ_LEN, DIM)


# speedup vs baseline: 1.0839x; 1.0839x over previous
import functools

import jax
import jax.numpy as jnp
from jax import lax
from jax.experimental import pallas as pl
from jax.experimental.pallas import tpu as pltpu
from jax.experimental.pallas import tpu_sc as plsc

VOCAB = 1000000
DIM = 64
SEQ_LEN = 200
BATCH = 4096

NUM_CORES = 2
NUM_SUBCORES = 16
NW = NUM_CORES * NUM_SUBCORES
ROWS = BATCH * SEQ_LEN
ROWS_PER_W = ROWS // NW
CHUNK = 4 * SEQ_LEN
CHUNKS_PER_W = ROWS_PER_W // CHUNK
GATHER = 100
GPC = CHUNK // GATHER
IDX_COLS = GATHER
LANES = 16
PAIRS = CHUNKS_PER_W // 2


def _pos_encoding():
    even_i = jnp.arange(0, DIM, 2).astype(jnp.float32)
    denominator = jnp.power(10000.0, even_i / DIM)
    position = jnp.arange(SEQ_LEN).reshape(SEQ_LEN, 1).astype(jnp.float32)
    even_pe = jnp.sin(position / denominator)
    odd_pe = jnp.cos(position / denominator)
    return jnp.stack([even_pe, odd_pe], axis=2).reshape(SEQ_LEN, DIM)


def _body(idx_hbm, pe_hbm, tab_hbm, out_hbm,
          idx0, idx1, rows0, rows1, pe_v,
          gsem0, gsem1, wsem0, wsem1):
    wid = lax.axis_index("s") * NUM_CORES + lax.axis_index("c")
    pltpu.sync_copy(pe_hbm, pe_v)
    idx_row0 = wid * (ROWS_PER_W // IDX_COLS)
    out_row0 = wid * ROWS_PER_W

    def stage_idx(c, idx_v):
        pltpu.sync_copy(
            idx_hbm.at[pl.ds(idx_row0 + c * GPC, GPC)], idx_v)

    def fire_gathers(idx_v, rows_v, gsem):
        for j in range(GPC):
            pltpu.make_async_copy(
                tab_hbm.at[idx_v.at[j]],
                rows_v.at[pl.ds(j * GATHER, GATHER)],
                gsem,
            ).start()

    def wait_gathers(idx_v, rows_v, gsem):
        for j in range(GPC):
            pltpu.make_async_copy(
                tab_hbm.at[idx_v.at[j]],
                rows_v.at[pl.ds(j * GATHER, GATHER)],
                gsem,
            ).wait()

    def add_pe(rows_v):
        def add_body(s, carry):
            for rep in range(CHUNK // SEQ_LEN):
                row = rep * SEQ_LEN + s
                for d in range(DIM // LANES):
                    sl = pl.ds(d * LANES, LANES)
                    rows_v[row, sl] = rows_v[row, sl] + pe_v[s, sl]
            return carry
        lax.fori_loop(0, SEQ_LEN, add_body, 0)

    def fire_wb(c, rows_v, wsem):
        pltpu.make_async_copy(
            rows_v, out_hbm.at[pl.ds(out_row0 + c * CHUNK, CHUNK)], wsem
        ).start()

    def drain_wb(rows_v, wsem):
        pltpu.make_async_copy(
            rows_v, out_hbm.at[pl.ds(out_row0, CHUNK)], wsem
        ).wait()

    stage_idx(0, idx0)
    fire_gathers(idx0, rows0, gsem0)

    def pair_body(g, carry):
        c = 2 * g
        stage_idx(c + 1, idx1)

        @pl.when(g >= 1)
        def _():
            drain_wb(rows1, wsem1)
        fire_gathers(idx1, rows1, gsem1)
        wait_gathers(idx0, rows0, gsem0)
        add_pe(rows0)
        fire_wb(c, rows0, wsem0)

        @pl.when(g < PAIRS - 1)
        def _():
            stage_idx(c + 2, idx0)
            drain_wb(rows0, wsem0)
            fire_gathers(idx0, rows0, gsem0)
        wait_gathers(idx1, rows1, gsem1)
        add_pe(rows1)
        fire_wb(c + 1, rows1, wsem1)
        return carry

    lax.fori_loop(0, PAIRS, pair_body, 0)

    drain_wb(rows0, wsem0)
    drain_wb(rows1, wsem1)


_emb_call = functools.partial(
    pl.kernel,
    mesh=plsc.VectorSubcoreMesh(core_axis_name="c", subcore_axis_name="s"),
    out_type=jax.ShapeDtypeStruct((ROWS, DIM), jnp.float32),
    scratch_types=[
        pltpu.VMEM((GPC, IDX_COLS), jnp.int32),
        pltpu.VMEM((GPC, IDX_COLS), jnp.int32),
        pltpu.VMEM((CHUNK, DIM), jnp.float32),
        pltpu.VMEM((CHUNK, DIM), jnp.float32),
        pltpu.VMEM((SEQ_LEN, DIM), jnp.float32),
        pltpu.SemaphoreType.DMA,
        pltpu.SemaphoreType.DMA,
        pltpu.SemaphoreType.DMA,
        pltpu.SemaphoreType.DMA,
    ],
    compiler_params=pltpu.CompilerParams(use_tc_tiling_on_sc=False),
)(_body)


@jax.jit
def kernel(inputs, word_embedding):
    idx = inputs.astype(jnp.int32).reshape(ROWS // IDX_COLS, IDX_COLS)
    pe = _pos_encoding()
    out = _emb_call(idx, pe, word_embedding)
    return out.reshape(BATCH, SEQ_LEN, DIM)
